# baseline (speedup 1.0000x reference)
import functools

import jax
import jax.numpy as jnp
from jax import lax
from jax.experimental import pallas as pl
from jax.experimental.pallas import tpu as pltpu


def _pick_h_blk(hidden):
    for blk in (4096, 2048, 1024, 512, 256, 128):
        if hidden % blk == 0:
            return blk
    return None


def _kstream_kernel(x_ref, w_ref, o_ref):
    k = pl.program_id(0)
    part = lax.dot_general(
        x_ref[...],
        w_ref[...],
        dimension_numbers=(((1,), (1,)), ((), ())),
        preferred_element_type=jnp.float32,
    )

    @pl.when(k == 0)
    def _():
        o_ref[...] = part

    @pl.when(k != 0)
    def _():
        o_ref[...] += part


def _single_shot(x_ref, w_ref, o_ref):
    o_ref[...] = lax.dot_general(
        x_ref[...],
        w_ref[...],
        dimension_numbers=(((1,), (1,)), ((), ())),
        preferred_element_type=jnp.float32,
    )


def _linear_impl(x, weight):
    batch, seq, hidden = x.shape
    out_features, hidden_w = weight.shape
    assert hidden == hidden_w
    m = batch * seq
    x2d = x.reshape(m, hidden)

    cost = pl.CostEstimate(
        flops=2 * m * out_features * hidden,
        transcendentals=0,
        bytes_accessed=(m * hidden + out_features * hidden) * 4
        + m * out_features * 4,
    )

    h_blk = _pick_h_blk(hidden)
    if h_blk is None or hidden // h_blk < 2:
        y2d = pl.pallas_call(
            _single_shot,
            out_shape=jax.ShapeDtypeStruct((m, out_features), jnp.float32),
            in_specs=[
                pl.BlockSpec(memory_space=pltpu.MemorySpace.VMEM),
                pl.BlockSpec(memory_space=pltpu.MemorySpace.VMEM),
            ],
            out_specs=pl.BlockSpec(memory_space=pltpu.MemorySpace.VMEM),
            cost_estimate=cost,
        )(x2d, weight)
        return y2d.astype(x.dtype).reshape(batch, seq, out_features)

    nk = hidden // h_blk
    footprint = 2 * (m + out_features) * h_blk * 4 + m * out_features * 4
    vmem_limit = min(max(int(footprint * 1.5), 8 << 20), 48 << 20)

    y2d = pl.pallas_call(
        _kstream_kernel,
        out_shape=jax.ShapeDtypeStruct((m, out_features), jnp.float32),
        grid=(nk,),
        in_specs=[
            pl.BlockSpec((m, h_blk), lambda k: (0, k)),
            pl.BlockSpec((out_features, h_blk), lambda k: (0, k)),
        ],
        out_specs=pl.BlockSpec((m, out_features), lambda k: (0, 0)),
        compiler_params=pltpu.CompilerParams(
            dimension_semantics=("arbitrary",),
            vmem_limit_bytes=vmem_limit,
        ),
        cost_estimate=cost,
    )(x2d, weight)
    return y2d.astype(x.dtype).reshape(batch, seq, out_features)


kernel = jax.jit(_linear_impl)


# speedup vs baseline: 1.7194x; 1.7194x over previous
import functools

import jax
import jax.numpy as jnp
from jax import lax
from jax.experimental import pallas as pl
from jax.experimental.pallas import tpu as pltpu


def _pick_h_blk(hidden):
    for blk in (4096, 2048, 1024, 512, 256, 128):
        if hidden % blk == 0:
            return blk
    return None


def _kstream_kernel(x_ref, w_ref, o_ref):
    k = pl.program_id(0)
    part = lax.dot_general(
        x_ref[...],
        w_ref[...],
        dimension_numbers=(((2,), (1,)), ((), ())),
        preferred_element_type=jnp.float32,
    )

    @pl.when(k == 0)
    def _():
        o_ref[...] = part

    @pl.when(k != 0)
    def _():
        o_ref[...] += part


def _single_shot(x_ref, w_ref, o_ref):
    o_ref[...] = lax.dot_general(
        x_ref[...],
        w_ref[...],
        dimension_numbers=(((2,), (1,)), ((), ())),
        preferred_element_type=jnp.float32,
    )


def _linear_impl(x, weight):
    batch, seq, hidden = x.shape
    out_features, hidden_w = weight.shape
    assert hidden == hidden_w
    m = batch * seq

    cost = pl.CostEstimate(
        flops=2 * m * out_features * hidden,
        transcendentals=0,
        bytes_accessed=(m * hidden + out_features * hidden) * 4
        + m * out_features * 4,
    )

    h_blk = _pick_h_blk(hidden)
    if h_blk is None or hidden // h_blk < 2:
        y = pl.pallas_call(
            _single_shot,
            out_shape=jax.ShapeDtypeStruct((batch, seq, out_features), jnp.float32),
            in_specs=[
                pl.BlockSpec(memory_space=pltpu.MemorySpace.VMEM),
                pl.BlockSpec(memory_space=pltpu.MemorySpace.VMEM),
            ],
            out_specs=pl.BlockSpec(memory_space=pltpu.MemorySpace.VMEM),
            cost_estimate=cost,
        )(x, weight)
        return y.astype(x.dtype)

    nk = hidden // h_blk
    footprint = 2 * (m + out_features) * h_blk * 4 + m * out_features * 4
    vmem_limit = min(max(int(footprint * 1.5), 8 << 20), 48 << 20)

    y = pl.pallas_call(
        _kstream_kernel,
        out_shape=jax.ShapeDtypeStruct((batch, seq, out_features), jnp.float32),
        grid=(nk,),
        in_specs=[
            pl.BlockSpec((batch, seq, h_blk), lambda k: (0, 0, k)),
            pl.BlockSpec((out_features, h_blk), lambda k: (0, k)),
        ],
        out_specs=pl.BlockSpec((batch, seq, out_features), lambda k: (0, 0, 0)),
        compiler_params=pltpu.CompilerParams(
            dimension_semantics=("arbitrary",),
            vmem_limit_bytes=vmem_limit,
        ),
        cost_estimate=cost,
    )(x, weight)
    return y.astype(x.dtype)


kernel = jax.jit(_linear_impl)


# speedup vs baseline: 3.6441x; 2.1194x over previous
import functools

import jax
import jax.numpy as jnp
from jax import lax
from jax.experimental import pallas as pl
from jax.experimental.pallas import tpu as pltpu


def _pick_h_blk(hidden):
    for blk in (4096, 2048, 1024, 512, 256, 128):
        if hidden % blk == 0:
            return blk
    return None


def _kstream_kernel(x_ref, w_ref, o_ref):
    k = pl.program_id(0)
    part = lax.dot_general(
        x_ref[...],
        w_ref[...],
        dimension_numbers=(((2,), (1,)), ((), ())),
        preferred_element_type=jnp.float32,
    )

    @pl.when(k == 0)
    def _():
        o_ref[...] = part

    @pl.when(k != 0)
    def _():
        o_ref[...] += part


def _single_shot(x_ref, w_ref, o_ref):
    o_ref[...] = lax.dot_general(
        x_ref[...],
        w_ref[...],
        dimension_numbers=(((2,), (1,)), ((), ())),
        preferred_element_type=jnp.float32,
    )


def _linear_impl(x, weight):
    batch, seq, hidden = x.shape
    out_features, hidden_w = weight.shape
    assert hidden == hidden_w
    m = batch * seq

    xt = jnp.transpose(x, (1, 0, 2))

    cost = pl.CostEstimate(
        flops=2 * m * out_features * hidden,
        transcendentals=0,
        bytes_accessed=(m * hidden + out_features * hidden) * 4
        + m * out_features * 4,
    )

    h_blk = _pick_h_blk(hidden)
    if h_blk is None or hidden // h_blk < 2:
        yt = pl.pallas_call(
            _single_shot,
            out_shape=jax.ShapeDtypeStruct((seq, batch, out_features), jnp.float32),
            in_specs=[
                pl.BlockSpec(memory_space=pltpu.MemorySpace.VMEM),
                pl.BlockSpec(memory_space=pltpu.MemorySpace.VMEM),
            ],
            out_specs=pl.BlockSpec(memory_space=pltpu.MemorySpace.VMEM),
            cost_estimate=cost,
        )(xt, weight)
        return jnp.transpose(yt, (1, 0, 2)).astype(x.dtype)

    nk = hidden // h_blk
    footprint = 2 * (m + out_features) * h_blk * 4 + m * out_features * 4
    vmem_limit = min(max(int(footprint * 1.5), 8 << 20), 48 << 20)

    yt = pl.pallas_call(
        _kstream_kernel,
        out_shape=jax.ShapeDtypeStruct((seq, batch, out_features), jnp.float32),
        grid=(nk,),
        in_specs=[
            pl.BlockSpec((seq, batch, h_blk), lambda k: (0, 0, k)),
            pl.BlockSpec((out_features, h_blk), lambda k: (0, k)),
        ],
        out_specs=pl.BlockSpec((seq, batch, out_features), lambda k: (0, 0, 0)),
        compiler_params=pltpu.CompilerParams(
            dimension_semantics=("arbitrary",),
            vmem_limit_bytes=vmem_limit,
        ),
        cost_estimate=cost,
    )(xt, weight)
    return jnp.transpose(yt, (1, 0, 2)).astype(x.dtype)


kernel = jax.jit(_linear_impl)


# speedup vs baseline: 5.0338x; 1.3813x over previous
import functools

import jax
import jax.numpy as jnp
from jax import lax
from jax.experimental import pallas as pl
from jax.experimental.pallas import tpu as pltpu


def _pick_h_blk(hidden):
    for blk in (4096, 2048, 1024, 512, 256, 128):
        if hidden % blk == 0:
            return blk
    return None


def _kstream_kernel(x_ref, w_ref, o_ref):
    k = pl.program_id(0)
    part = lax.dot_general(
        x_ref[...],
        w_ref[...],
        dimension_numbers=(((2,), (1,)), ((), ())),
        preferred_element_type=jnp.float32,
    )

    @pl.when(k == 0)
    def _():
        o_ref[...] = part

    @pl.when(k != 0)
    def _():
        o_ref[...] += part


def _single_shot(x_ref, w_ref, o_ref):
    o_ref[...] = lax.dot_general(
        x_ref[...],
        w_ref[...],
        dimension_numbers=(((2,), (1,)), ((), ())),
        preferred_element_type=jnp.float32,
    )


def _linear_impl(x, weight):
    batch, seq, hidden = x.shape
    out_features, hidden_w = weight.shape
    assert hidden == hidden_w
    m = batch * seq

    xt = jnp.transpose(x, (1, 0, 2))

    cost = pl.CostEstimate(
        flops=2 * m * out_features * hidden,
        transcendentals=0,
        bytes_accessed=(m * hidden + out_features * hidden) * 4
        + m * out_features * 4,
    )

    h_blk = _pick_h_blk(hidden)
    if h_blk is None or hidden // h_blk < 2:
        yt = pl.pallas_call(
            _single_shot,
            out_shape=jax.ShapeDtypeStruct((seq, batch, out_features), jnp.float32),
            in_specs=[
                pl.BlockSpec(memory_space=pltpu.MemorySpace.VMEM),
                pl.BlockSpec(memory_space=pltpu.MemorySpace.VMEM),
            ],
            out_specs=pl.BlockSpec(memory_space=pltpu.MemorySpace.VMEM),
            cost_estimate=cost,
        )(xt, weight)
        return jnp.transpose(yt, (1, 0, 2)).astype(x.dtype)

    nk = hidden // h_blk
    vmem_limit = 60000 * 1024

    yt = pl.pallas_call(
        _kstream_kernel,
        out_shape=jax.ShapeDtypeStruct((seq, batch, out_features), jnp.float32),
        grid=(nk,),
        in_specs=[
            pl.BlockSpec((seq, batch, h_blk), lambda k: (0, 0, k)),
            pl.BlockSpec((out_features, h_blk), lambda k: (0, k)),
        ],
        out_specs=pl.BlockSpec((seq, batch, out_features), lambda k: (0, 0, 0)),
        compiler_params=pltpu.CompilerParams(
            dimension_semantics=("arbitrary",),
            vmem_limit_bytes=vmem_limit,
        ),
        cost_estimate=cost,
    )(xt, weight)
    return jnp.transpose(yt, (1, 0, 2)).astype(x.dtype)


kernel = jax.jit(_linear_impl)


# speedup vs baseline: 5.6458x; 1.1216x over previous
import functools

import jax
import jax.numpy as jnp
from jax import lax
from jax.experimental import pallas as pl
from jax.experimental.pallas import tpu as pltpu


def _pick_h_blk(hidden):
    for blk in (8192, 4096, 2048, 1024, 512, 256, 128):
        if hidden % blk == 0:
            return blk
    return None


def _kstream_kernel(x_ref, w_ref, o_ref):
    k = pl.program_id(0)
    part = lax.dot_general(
        x_ref[...],
        w_ref[...],
        dimension_numbers=(((2,), (1,)), ((), ())),
        preferred_element_type=jnp.float32,
    )

    @pl.when(k == 0)
    def _():
        o_ref[...] = part

    @pl.when(k != 0)
    def _():
        o_ref[...] += part


def _single_shot(x_ref, w_ref, o_ref):
    o_ref[...] = lax.dot_general(
        x_ref[...],
        w_ref[...],
        dimension_numbers=(((2,), (1,)), ((), ())),
        preferred_element_type=jnp.float32,
    )


def _linear_impl(x, weight):
    batch, seq, hidden = x.shape
    out_features, hidden_w = weight.shape
    assert hidden == hidden_w
    m = batch * seq

    xt = jnp.transpose(x, (1, 0, 2))

    cost = pl.CostEstimate(
        flops=2 * m * out_features * hidden,
        transcendentals=0,
        bytes_accessed=(m * hidden + out_features * hidden) * 4
        + m * out_features * 4,
    )

    h_blk = _pick_h_blk(hidden)
    if h_blk is None or hidden // h_blk < 2:
        yt = pl.pallas_call(
            _single_shot,
            out_shape=jax.ShapeDtypeStruct((seq, batch, out_features), jnp.float32),
            in_specs=[
                pl.BlockSpec(memory_space=pltpu.MemorySpace.VMEM),
                pl.BlockSpec(memory_space=pltpu.MemorySpace.VMEM),
            ],
            out_specs=pl.BlockSpec(memory_space=pltpu.MemorySpace.VMEM),
            cost_estimate=cost,
        )(xt, weight)
        return jnp.transpose(yt, (1, 0, 2)).astype(x.dtype)

    nk = hidden // h_blk
    vmem_limit = 60000 * 1024

    yt = pl.pallas_call(
        _kstream_kernel,
        out_shape=jax.ShapeDtypeStruct((seq, batch, out_features), jnp.float32),
        grid=(nk,),
        in_specs=[
            pl.BlockSpec((seq, batch, h_blk), lambda k: (0, 0, k)),
            pl.BlockSpec((out_features, h_blk), lambda k: (0, k)),
        ],
        out_specs=pl.BlockSpec((seq, batch, out_features), lambda k: (0, 0, 0)),
        compiler_params=pltpu.CompilerParams(
            dimension_semantics=("arbitrary",),
            vmem_limit_bytes=vmem_limit,
        ),
        cost_estimate=cost,
    )(xt, weight)
    return jnp.transpose(yt, (1, 0, 2)).astype(x.dtype)


kernel = jax.jit(_linear_impl)
